# speedup vs baseline: 9.2261x; 9.2261x over previous
import functools

import jax
import jax.numpy as jnp
from jax import lax
from jax.experimental import pallas as pl
from jax.experimental.pallas import tpu as pltpu
from jax.experimental.pallas import tpu_sc as plsc

N_ROWS = 4096 * 200
D = 128
C = 128
NBUF = 4
NW = 32
G_PER_W = N_ROWS // (C * NW)
NG = G_PER_W // NBUF
NC = 2


def _make_gather():
    mesh = plsc.VectorSubcoreMesh(core_axis_name="c", subcore_axis_name="s")

    @functools.partial(
        pl.kernel,
        mesh=mesh,
        out_type=jax.ShapeDtypeStruct((N_ROWS, D), jnp.float32),
        scratch_types=[
            pltpu.VMEM((G_PER_W, C), jnp.int32),
            pltpu.VMEM((NBUF, C, D), jnp.float32),
            pltpu.SemaphoreType.DMA((NBUF,)),
            pltpu.SemaphoreType.DMA((NBUF,)),
        ],
    )
    def gather_kernel(idx_hbm, table_hbm, out_hbm, idx_v, bufs, gsem, ssem):
        wid = lax.axis_index("s") * NC + lax.axis_index("c")
        cbase = wid * G_PER_W

        pltpu.sync_copy(idx_hbm.at[pl.ds(cbase, G_PER_W)], idx_v)

        def gather_fire(j, b):
            pltpu.async_copy(table_hbm.at[idx_v.at[j]], bufs.at[b], gsem.at[b])

        def gather_wait(j, b):
            pltpu.make_async_copy(
                table_hbm.at[idx_v.at[j]], bufs.at[b], gsem.at[b]).wait()

        def store_fire(j, b):
            pltpu.async_copy(
                bufs.at[b], out_hbm.at[pl.ds((cbase + j) * C, C)], ssem.at[b])

        def store_wait(j, b):
            pltpu.make_async_copy(
                bufs.at[b], out_hbm.at[pl.ds((cbase + j) * C, C)],
                ssem.at[b]).wait()

        for b in range(NBUF):
            gather_fire(b, b)

        def group(g, carry):
            for b in range(NBUF):
                j = g * NBUF + b
                gather_wait(j, b)
                store_fire(j, b)
                store_wait(j, b)
                gather_fire(j + NBUF, b)
            return carry

        lax.fori_loop(0, NG - 1, group, 0)

        for b in range(NBUF):
            j = (NG - 1) * NBUF + b
            gather_wait(j, b)
            store_fire(j, b)
            store_wait(j, b)

    return gather_kernel


_gather = _make_gather()


def kernel(input, weight):
    idx = input.reshape(N_ROWS // C, C).astype(jnp.int32)
    out = _gather(idx, weight)
    return out.reshape(input.shape + (weight.shape[1],))


# speedup vs baseline: 9.3110x; 1.0092x over previous
import functools

import jax
import jax.numpy as jnp
from jax import lax
from jax.experimental import pallas as pl
from jax.experimental.pallas import tpu as pltpu
from jax.experimental.pallas import tpu_sc as plsc

N_ROWS = 4096 * 200
D = 128
C = 128
NBUF = 6
PRE = 4
NW = 32
G_PER_W = N_ROWS // (C * NW)
NC = 2


def _make_gather():
    mesh = plsc.VectorSubcoreMesh(core_axis_name="c", subcore_axis_name="s")

    @functools.partial(
        pl.kernel,
        mesh=mesh,
        out_type=jax.ShapeDtypeStruct((N_ROWS, D), jnp.float32),
        scratch_types=[
            pltpu.VMEM((G_PER_W, C), jnp.int32),
            pltpu.VMEM((NBUF, C, D), jnp.float32),
            pltpu.SemaphoreType.DMA((NBUF,)),
            pltpu.SemaphoreType.DMA((NBUF,)),
        ],
    )
    def gather_kernel(idx_hbm, table_hbm, out_hbm, idx_v, bufs, gsem, ssem):
        wid = lax.axis_index("s") * NC + lax.axis_index("c")
        cbase = wid * G_PER_W

        pltpu.sync_copy(idx_hbm.at[pl.ds(cbase, G_PER_W)], idx_v)

        def gather_fire(j, b):
            pltpu.async_copy(table_hbm.at[idx_v.at[j]], bufs.at[b], gsem.at[b])

        def gather_wait(j, b):
            pltpu.make_async_copy(
                table_hbm.at[idx_v.at[j]], bufs.at[b], gsem.at[b]).wait()

        def store_fire(j, b):
            pltpu.async_copy(
                bufs.at[b], out_hbm.at[pl.ds((cbase + j) * C, C)], ssem.at[b])

        def store_wait(j, b):
            pltpu.make_async_copy(
                bufs.at[b], out_hbm.at[pl.ds((cbase + j) * C, C)],
                ssem.at[b]).wait()

        for j in range(PRE):
            gather_fire(j, j)

        def step(j, carry):
            b = lax.rem(j, NBUF)
            gather_wait(j, b)
            store_fire(j, b)
            f = j + PRE
            fb = lax.rem(f, NBUF)

            @pl.when(f < G_PER_W)
            def _():
                @pl.when(j >= NBUF - PRE)
                def _():
                    store_wait(f - NBUF, fb)

                gather_fire(f, fb)

            return carry

        lax.fori_loop(0, G_PER_W, step, 0)

        for j in range(G_PER_W - NBUF, G_PER_W):
            store_wait(j, j % NBUF)

    return gather_kernel


_gather = _make_gather()


def kernel(input, weight):
    idx = input.reshape(N_ROWS // C, C).astype(jnp.int32)
    out = _gather(idx, weight)
    return out.reshape(input.shape + (weight.shape[1],))
